# baseline (speedup 1.0000x reference)
import functools

import jax
import jax.numpy as jnp
import numpy as np
from jax import lax
from jax.experimental import pallas as pl
from jax.experimental.pallas import tpu as pltpu
from jax.experimental.pallas import tpu_sc as plsc

MAX_DAYS = 365.0
D = 64
NCORES = 2
NSUB = 16
NW = NCORES * NSUB
BLK = 256


def _group_compute(g, c624_v, ab_v, bpv, g_chunks, b_chunks, ci_v, u_v,
                   out_v):
    f32 = jnp.float32
    i32 = jnp.int32
    u16 = u_v[pl.ds(g * 16, 16)]
    ci16 = ci_v[pl.ds(g * 16, 16)] * D
    k16 = jnp.zeros((16,), i32)
    one16 = jnp.full((16,), 1, i32)
    zero16 = jnp.zeros((16,), i32)
    for j in range(16):
        bsplat = jnp.full((16,), bpv[j], f32)
        k16 = k16 + jnp.where(u16 > bsplat, one16, zero16)
    a16 = k16 * D
    b16 = a16 + 17 * D
    lane = jnp.arange(16, dtype=i32)
    vvec = jnp.zeros((16,), f32)
    for l in range(16):
        usplat = jnp.full((16,), u16[l], f32)
        arow = a16[l]
        brow = b16[l]
        crow = ci16[l]
        obase = (g * 16 + l) * D
        acc = None
        for ch in range(4):
            cval = c624_v[pl.ds(crow + 16 * ch, 16)]
            aval = ab_v[pl.ds(arow + 16 * ch, 16)]
            bval = ab_v[pl.ds(brow + 16 * ch, 16)]
            cc = cval + aval * usplat + bval
            out_v[pl.ds(obase + 16 * ch, 16)] = cc
            acc = cc * cc if acc is None else acc + cc * cc
        s = acc
        for step in (8, 4, 2, 1):
            perm = jnp.arange(16, dtype=jnp.int32) ^ step
            s = s + s.at[perm].get(mode="promise_in_bounds")
        vvec = jnp.where(lane == l, s, vvec)
    vvec = vvec * (1.0 / D) + 1e-5
    lo = vvec < 1.0
    r = jnp.where(lo, jnp.full((16,), 65536.0, f32), jnp.full((16,), 1.0, f32))
    m = jnp.where(lo, vvec * 4294967296.0, vvec)
    for e in (32, 16, 8, 4, 2, 1):
        cond = m >= (2.0 ** (2 * e))
        m = jnp.where(cond, m * (2.0 ** (-2 * e)), m)
        r = jnp.where(cond, r * (2.0 ** (-e)), r)
    y = (0.0446324 * m - 0.37001683) * m + 1.27889388
    for _ in range(3):
        y = y * (1.5 - (0.5 * m) * y * y)
    y = y * r
    for l in range(16):
        ysplat = jnp.full((16,), y[l], f32)
        obase = (g * 16 + l) * D
        for ch in range(4):
            cc = out_v[pl.ds(obase + 16 * ch, 16)]
            out_v[pl.ds(obase + 16 * ch, 16)] = (
                cc * (ysplat * g_chunks[ch]) + b_chunks[ch])


def _sc_kernel_body(ntok_per_tile, c624_hbm, ab_hbm, bp_hbm, gb_hbm,
                    ci_hbm, u_hbm, out_hbm,
                    c624_v, ab_v, bp_v, gb_v, ci_v, u_v, out_v):
    cid = lax.axis_index("c")
    sid = lax.axis_index("s")
    wid = sid * NCORES + cid
    pltpu.sync_copy(c624_hbm, c624_v)
    pltpu.sync_copy(ab_hbm, ab_v)
    pltpu.sync_copy(bp_hbm, bp_v)
    pltpu.sync_copy(gb_hbm, gb_v)
    bpv = bp_v[:]
    g_chunks = [gb_v[pl.ds(16 * ch, 16)] for ch in range(4)]
    b_chunks = [gb_v[pl.ds(D + 16 * ch, 16)] for ch in range(4)]
    base0 = wid * ntok_per_tile
    nblk = ntok_per_tile // BLK

    def blk_body(blk, carry):
        base = base0 + blk * BLK
        pltpu.sync_copy(ci_hbm.at[pl.ds(base, BLK)], ci_v)
        pltpu.sync_copy(u_hbm.at[pl.ds(base, BLK)], u_v)

        def grp_body(g, c):
            _group_compute(g, c624_v, ab_v, bpv, g_chunks, b_chunks,
                           ci_v, u_v, out_v)
            return c

        lax.fori_loop(0, BLK // 16, grp_body, 0)
        pltpu.sync_copy(out_v, out_hbm.at[pl.ds(base * D, BLK * D)])
        return carry

    lax.fori_loop(0, nblk, blk_body, 0)


def kernel(cat_id, amount_bucket, channel_id, days_delta,
           cat_table, bucket_table, channel_table,
           days_w, days_b, proj_w, proj_b, ln_g, ln_b):
    B, S = cat_id.shape
    M = B * S
    f32 = jnp.float32
    ntok_per_tile = M // NW

    catP = cat_table @ proj_w[:, 0:16].T
    bktP = bucket_table @ proj_w[:, 16:24].T
    chnP = channel_table @ proj_w[:, 40:48].T
    c624 = (catP[:, None, None, :] + bktP[None, :, None, :]
            + chnP[None, None, :, :] + proj_b).reshape(624, D)
    c624 = c624 - jnp.mean(c624, axis=1, keepdims=True)

    w = days_w.astype(f32)
    b = days_b.astype(f32)
    bp = jnp.where(w != 0.0, -b / jnp.where(w != 0.0, w, 1.0), jnp.inf)
    cj = jnp.sum(bp[:, None] <= bp[None, :], axis=0)
    ks = jnp.arange(17)[:, None]
    act = jnp.where(w[None, :] > 0.0, ks >= cj[None, :],
                    jnp.where(w[None, :] < 0.0, ks < cj[None, :],
                              b[None, :] > 0.0)).astype(f32)
    wdT = proj_w[:, 24:40].T
    A = (act * w[None, :]) @ wdT
    Bm = (act * b[None, :]) @ wdT
    A = A - jnp.mean(A, axis=1, keepdims=True)
    Bm = Bm - jnp.mean(Bm, axis=1, keepdims=True)
    ab = jnp.concatenate([A, Bm], axis=0)
    gb = jnp.concatenate([ln_g.astype(f32), ln_b.astype(f32)])

    ci = (cat_id * 48 + amount_bucket * 8 + channel_id).reshape(M)
    u = (days_delta * (1.0 / MAX_DAYS)).astype(f32).reshape(M)

    mesh = plsc.VectorSubcoreMesh(core_axis_name="c", subcore_axis_name="s")
    kern = pl.kernel(
        functools.partial(_sc_kernel_body, ntok_per_tile),
        mesh=mesh,
        out_type=jax.ShapeDtypeStruct((M * D,), f32),
        scratch_types=[
            pltpu.VMEM((624 * D,), f32),
            pltpu.VMEM((34 * D,), f32),
            pltpu.VMEM((16,), f32),
            pltpu.VMEM((2 * D,), f32),
            pltpu.VMEM((BLK,), jnp.int32),
            pltpu.VMEM((BLK,), f32),
            pltpu.VMEM((BLK * D,), f32),
        ],
    )
    out = kern(c624.reshape(624 * D), ab.reshape(34 * D), bp, gb, ci, u)
    return out.reshape(B, S, D)
